# Initial kernel scaffold; baseline (speedup 1.0000x reference)
import jax
import jax.numpy as jnp
from jax.experimental import pallas as pl


def kernel(logits, locs, scales, discrete, continuous):
    raise NotImplementedError("write your pallas kernel here")


# speedup vs baseline: 1.4659x; 1.4659x over previous
import functools

import jax
import jax.numpy as jnp
from jax import lax
from jax.experimental import pallas as pl
from jax.experimental.pallas import tpu as pltpu
from jax.experimental.pallas import tpu_sc as plsc

_SUPPORT = 1000000
_BATCH = 16384
_ROWS = 128
_COLS = 128
_NW = 32
_RPW = _ROWS // _NW
_HALF_LOG_2PI = 0.9189385332046727


def _sc_gather_body(disc_hbm, logits_hbm, locs_hbm, scales_hbm,
                    glog_hbm, gloc_hbm, gscale_hbm,
                    idx_v, g1, g2, g3, sem):
    wid = lax.axis_index("s") * 2 + lax.axis_index("c")
    base = wid * _RPW
    pltpu.sync_copy(disc_hbm.at[pl.ds(base, _RPW)], idx_v)
    copies = []
    for j in range(_RPW):
        copies.append(pltpu.async_copy(logits_hbm.at[idx_v.at[j]], g1.at[j], sem))
        copies.append(pltpu.async_copy(locs_hbm.at[idx_v.at[j]], g2.at[j], sem))
        copies.append(pltpu.async_copy(scales_hbm.at[idx_v.at[j]], g3.at[j], sem))
    for c in copies:
        c.wait()
    pltpu.sync_copy(g1, glog_hbm.at[pl.ds(base, _RPW)])
    pltpu.sync_copy(g2, gloc_hbm.at[pl.ds(base, _RPW)])
    pltpu.sync_copy(g3, gscale_hbm.at[pl.ds(base, _RPW)])


def _sc_gather(disc2, logits, locs, scales):
    mesh = plsc.VectorSubcoreMesh(core_axis_name="c", subcore_axis_name="s")
    f32 = jnp.float32
    out = jax.ShapeDtypeStruct((_ROWS, _COLS), f32)
    kfn = functools.partial(
        pl.kernel,
        mesh=mesh,
        out_type=[out, out, out],
        scratch_types=[
            pltpu.VMEM((_RPW, _COLS), jnp.int32),
            pltpu.VMEM((_RPW, _COLS), f32),
            pltpu.VMEM((_RPW, _COLS), f32),
            pltpu.VMEM((_RPW, _COLS), f32),
            pltpu.SemaphoreType.DMA,
        ],
    )(_sc_gather_body)
    return kfn(disc2, logits, locs, scales)


def _tc_body(logits_ref, glog_ref, gloc_ref, gscale_ref, cont_ref, out_ref):
    x = logits_ref[...]
    m = jnp.max(x)
    s = jnp.sum(jnp.exp(x - m))
    lse = m + jnp.log(s)
    gl = glog_ref[...]
    lo = gloc_ref[...]
    sc = gscale_ref[...]
    c = cont_ref[...]
    z = (c - lo) / sc
    out_ref[...] = gl - lse - 0.5 * z * z - jnp.log(sc) - _HALF_LOG_2PI


def kernel(logits, locs, scales, discrete, continuous):
    logits2 = logits.reshape(1000, 1000)
    disc2 = discrete.reshape(_ROWS, _COLS)
    cont2 = continuous.reshape(_ROWS, _COLS)
    glog, gloc, gscale = _sc_gather(disc2, logits, locs, scales)
    out2 = pl.pallas_call(
        _tc_body,
        out_shape=jax.ShapeDtypeStruct((_ROWS, _COLS), jnp.float32),
    )(logits2, glog, gloc, gscale, cont2)
    return out2.reshape(_BATCH)
